# baseline (speedup 1.0000x reference)
import jax
import jax.numpy as jnp
from jax.experimental import pallas as pl
from jax.experimental.pallas import tpu as pltpu


def _fused_gate_kernel(x_ref, w1_ref, w2_ref, o_ref, *, inv_hw):
    x = x_ref[...]

    tot = jnp.sum(x, axis=-1, dtype=jnp.float32)
    mx = jnp.max(x, axis=-1).astype(jnp.float32)
    avg = tot * inv_hw

    dn1 = (((1,), (1,)), ((), ()))
    ha = jax.lax.dot_general(avg, w1_ref[...], dn1,
                             preferred_element_type=jnp.float32)
    hm = jax.lax.dot_general(mx, w1_ref[...], dn1,
                             preferred_element_type=jnp.float32)
    h = jnp.maximum(ha, 0.0) + jnp.maximum(hm, 0.0)

    logits = jax.lax.dot_general(h, w2_ref[...], dn1,
                                 preferred_element_type=jnp.float32)
    gate = jax.nn.sigmoid(logits).astype(o_ref.dtype)

    o_ref[...] = x * gate[:, :, None]


def kernel(x, w1, w2):
    B, C, H, W = x.shape
    hw = H * W
    hidden = w1.shape[0]
    dtype = x.dtype
    itemsize = jnp.dtype(dtype).itemsize

    x3 = x.reshape(B, C, hw)

    per_batch_bytes = C * hw * itemsize
    bt = 1
    for cand in (16, 8, 4, 2):
        if B % cand == 0 and (B // cand) % 2 == 0 \
                and cand * per_batch_bytes <= 8 * 1024 * 1024:
            bt = cand
            break
    grid = pl.cdiv(B, bt)

    cost = pl.CostEstimate(
        flops=int(2 * (2 * B) * C * hidden * 2 + 3 * B * C * hw),
        transcendentals=int(B * C),
        bytes_accessed=int(2 * B * C * hw * itemsize))

    out = pl.pallas_call(
        lambda x_ref, w1_ref, w2_ref, o_ref: _fused_gate_kernel(
            x_ref, w1_ref, w2_ref, o_ref, inv_hw=1.0 / hw),
        out_shape=jax.ShapeDtypeStruct((B, C, hw), dtype),
        grid=(grid,),
        in_specs=[
            pl.BlockSpec((bt, C, hw), lambda b: (b, 0, 0)),
            pl.BlockSpec((hidden, C), lambda b: (0, 0)),
            pl.BlockSpec((C, hidden), lambda b: (0, 0)),
        ],
        out_specs=pl.BlockSpec((bt, C, hw), lambda b: (b, 0, 0)),
        compiler_params=pltpu.CompilerParams(
            dimension_semantics=("parallel",)),
        cost_estimate=cost,
    )(x3, w1, w2)
    return out.reshape(B, C, H, W)


# speedup vs baseline: 1.0013x; 1.0013x over previous
import jax
import jax.numpy as jnp
from jax.experimental import pallas as pl
from jax.experimental.pallas import tpu as pltpu


def _fused_gate_kernel(x_ref, w1_ref, w2_ref, o_ref, *, inv_hw):
    x = x_ref[...]

    hw = x.shape[-1]
    xs = x
    xm = x
    while hw > 128 and hw % 2 == 0:
        hw //= 2
        xs = xs[..., :hw] + xs[..., hw:]
        xm = jnp.maximum(xm[..., :hw], xm[..., hw:])
    tot = jnp.sum(xs, axis=-1, dtype=jnp.float32)
    mx = jnp.max(xm, axis=-1).astype(jnp.float32)
    avg = tot * inv_hw

    dn = (((1,), (1,)), ((), ()))
    ha = jax.lax.dot_general(avg, w1_ref[...], dn,
                             preferred_element_type=jnp.float32)
    hm = jax.lax.dot_general(mx, w1_ref[...], dn,
                             preferred_element_type=jnp.float32)
    h = jnp.maximum(ha, 0.0) + jnp.maximum(hm, 0.0)

    logits = jax.lax.dot_general(h, w2_ref[...], dn,
                                 preferred_element_type=jnp.float32)
    gate = jax.nn.sigmoid(logits).astype(o_ref.dtype)

    o_ref[...] = x_ref[...] * gate[:, :, None]


def kernel(x, w1, w2):
    B, C, H, W = x.shape
    hw = H * W
    hidden = w1.shape[0]
    dtype = x.dtype
    itemsize = jnp.dtype(dtype).itemsize

    x3 = x.reshape(B, C, hw)

    per_batch_bytes = C * hw * itemsize
    bt = 1
    for cand in (16, 8, 4, 2):
        if B % cand == 0 and (B // cand) % 2 == 0 \
                and cand * per_batch_bytes <= 8 * 1024 * 1024:
            bt = cand
            break
    grid = pl.cdiv(B, bt)

    cost = pl.CostEstimate(
        flops=int(2 * (2 * B) * C * hidden * 2 + 3 * B * C * hw),
        transcendentals=int(B * C),
        bytes_accessed=int(2 * B * C * hw * itemsize))

    out = pl.pallas_call(
        lambda x_ref, w1_ref, w2_ref, o_ref: _fused_gate_kernel(
            x_ref, w1_ref, w2_ref, o_ref, inv_hw=1.0 / hw),
        out_shape=jax.ShapeDtypeStruct((B, C, hw), dtype),
        grid=(grid,),
        in_specs=[
            pl.BlockSpec((bt, C, hw), lambda b: (b, 0, 0)),
            pl.BlockSpec((hidden, C), lambda b: (0, 0)),
            pl.BlockSpec((C, hidden), lambda b: (0, 0)),
        ],
        out_specs=pl.BlockSpec((bt, C, hw), lambda b: (b, 0, 0)),
        compiler_params=pltpu.CompilerParams(
            dimension_semantics=("parallel",)),
        cost_estimate=cost,
    )(x3, w1, w2)
    return out.reshape(B, C, H, W)
